# baseline (speedup 1.0000x reference)
import functools

import jax
import jax.numpy as jnp
from jax import lax
from jax.experimental import pallas as pl
from jax.experimental.pallas import tpu as pltpu
from jax.experimental.pallas import tpu_sc as plsc

_N = 10000
_E = 160000
_DIN = 128
_DHID = 128
_DLAT = 32

_NC = 2
_NS = 16
_NW = _NC * _NS
_CHUNK = 128
_NCHUNK = 40
_NCHALF = 20
_EPAD = _NW * _CHUNK * _NCHUNK
_EHALF = _EPAD // 2
_NPAD = 10240
_RPT = _NPAD // _NS
_ZR = 64
_DEGW = 16


_NBUF = 2


def _make_prop(d, nchunk):
    mesh = plsc.VectorSubcoreMesh(core_axis_name="c", subcore_axis_name="s")

    @functools.partial(
        pl.kernel,
        out_type=jax.ShapeDtypeStruct((_NC * _NPAD, d), jnp.float32),
        mesh=mesh,
        scratch_types=[
            pltpu.VMEM((nchunk, _CHUNK), jnp.int32),
            pltpu.VMEM((nchunk, _CHUNK), jnp.int32),
            pltpu.VMEM_SHARED((_NPAD, d), jnp.float32),
        ]
        + [pltpu.VMEM((_CHUNK, d), jnp.float32)] * _NBUF
        + [pltpu.SemaphoreType.DMA] * _NBUF,
    )
    def prop(hws_hbm, src_hbm, dst_hbm, out_hbm, src_v, dst_v, acc,
             *ring_and_sems):
        rows = ring_and_sems[:_NBUF]
        sems = ring_and_sems[_NBUF:]
        c = lax.axis_index("c")
        s = lax.axis_index("s")
        wid = c * _NS + s
        zero = jnp.zeros((16,), jnp.float32)
        for i in range(_CHUNK):
            for k in range(d // 16):
                rows[0][i, pl.ds(k * 16, 16)] = zero
        for t in range(_RPT // _CHUNK):
            pltpu.sync_copy(rows[0], acc.at[pl.ds(s * _RPT + t * _CHUNK, _CHUNK)])
        pltpu.sync_copy(src_hbm.at[wid], src_v)
        pltpu.sync_copy(dst_hbm.at[wid], dst_v)
        plsc.subcore_barrier()

        for b in range(_NBUF - 1):
            pltpu.async_copy(hws_hbm.at[src_v.at[b]], rows[b], sems[b])

        def group(g, carry):
            j0 = g * _NBUF
            for b in range(_NBUF):
                j = j0 + b
                jn = j + _NBUF - 1
                bn = (_NBUF - 1 + b) % _NBUF

                @pl.when(jn < nchunk)
                def _():
                    pltpu.async_copy(hws_hbm.at[src_v.at[jn]], rows[bn], sems[bn])

                pltpu.make_async_copy(
                    hws_hbm.at[src_v.at[j]], rows[b], sems[b]
                ).wait()
                pltpu.sync_copy(rows[b], acc.at[dst_v.at[j]], add=True)
            return carry

        lax.fori_loop(0, nchunk // _NBUF, group, 0)
        plsc.subcore_barrier()
        pltpu.sync_copy(
            acc.at[pl.ds(s * _RPT, _RPT)],
            out_hbm.at[pl.ds(c * _NPAD + s * _RPT, _RPT)],
        )

    return prop


_prop128 = _make_prop(_DHID, _NCHUNK)


def _make_deg():
    mesh = plsc.VectorSubcoreMesh(core_axis_name="c", subcore_axis_name="s")

    @functools.partial(
        pl.kernel,
        out_type=jax.ShapeDtypeStruct((_NC * _NPAD, _DEGW), jnp.float32),
        mesh=mesh,
        scratch_types=[
            pltpu.VMEM((_NCHUNK, _CHUNK), jnp.int32),
            pltpu.VMEM((_CHUNK, _DEGW), jnp.float32),
            pltpu.VMEM((_ZR, _DEGW), jnp.float32),
            pltpu.VMEM_SHARED((_NPAD, _DEGW), jnp.float32),
        ],
    )
    def deg(dst_hbm, out_hbm, dst_v, ones_v, zbuf, acc):
        c = lax.axis_index("c")
        s = lax.axis_index("s")
        wid = c * _NS + s
        one = jnp.full((16,), 1.0, jnp.float32)
        zero = jnp.zeros((16,), jnp.float32)
        for i in range(_CHUNK):
            ones_v[i, :] = one
        for i in range(_ZR):
            zbuf[i, :] = zero
        for t in range(_RPT // _ZR):
            pltpu.sync_copy(zbuf, acc.at[pl.ds(s * _RPT + t * _ZR, _ZR)])
        pltpu.sync_copy(dst_hbm.at[wid], dst_v)
        plsc.subcore_barrier()

        def body(j, carry):
            pltpu.sync_copy(ones_v, acc.at[dst_v.at[j]], add=True)
            return carry

        lax.fori_loop(0, _NCHUNK, body, 0)
        plsc.subcore_barrier()
        pltpu.sync_copy(
            acc.at[pl.ds(s * _RPT, _RPT)],
            out_hbm.at[pl.ds(c * _NPAD + s * _RPT, _RPT)],
        )

    return deg


_deg = _make_deg()

_BN = 1000


def _dinv_of(dc_ref):
    deg = dc_ref[0, :, 0:1] + dc_ref[1, :, 0:1] + 1.0
    return lax.rsqrt(deg)


def _enc1(x, w, dc):
    def body(x_ref, w_ref, dc_ref, o_ref):
        hw = jnp.dot(x_ref[...], w_ref[...], preferred_element_type=jnp.float32)
        o_ref[...] = hw * _dinv_of(dc_ref)

    return pl.pallas_call(
        body,
        grid=(_N // _BN,),
        in_specs=[
            pl.BlockSpec((_BN, _DIN), lambda i: (i, 0)),
            pl.BlockSpec((_DIN, _DHID), lambda i: (0, 0)),
            pl.BlockSpec((_NC, _BN, _DEGW), lambda i: (0, i, 0)),
        ],
        out_specs=pl.BlockSpec((_BN, _DHID), lambda i: (i, 0)),
        out_shape=jax.ShapeDtypeStruct((_N, _DHID), jnp.float32),
    )(x, w, dc)


def _enc2(s1a, hws1, wcat, dc):
    def body(sa_ref, hws1_ref, w_ref, dc_ref, o_ref):
        dinv = _dinv_of(dc_ref)
        h = jnp.maximum(dinv * (sa_ref[0] + sa_ref[1] + hws1_ref[...]), 0.0)
        o_ref[...] = dinv * jnp.dot(
            h, w_ref[...], preferred_element_type=jnp.float32
        )

    return pl.pallas_call(
        body,
        grid=(_N // _BN,),
        in_specs=[
            pl.BlockSpec((_NC, _BN, _DHID), lambda i: (0, i, 0)),
            pl.BlockSpec((_BN, _DHID), lambda i: (i, 0)),
            pl.BlockSpec((_DHID, _DHID), lambda i: (0, 0)),
            pl.BlockSpec((_NC, _BN, _DEGW), lambda i: (0, i, 0)),
        ],
        out_specs=pl.BlockSpec((_BN, _DHID), lambda i: (i, 0)),
        out_shape=jax.ShapeDtypeStruct((_N, _DHID), jnp.float32),
    )(s1a, hws1, wcat, dc)


def _latent(s2a, hws2, dc, eps):
    def body(sa_ref, hws2_ref, dc_ref, eps_ref, mu_ref, ls_ref, z_ref):
        dinv = _dinv_of(dc_ref)
        agg = dinv * (
            sa_ref[0, :, : 2 * _DLAT]
            + sa_ref[1, :, : 2 * _DLAT]
            + hws2_ref[:, : 2 * _DLAT]
        )
        mu = agg[:, :_DLAT]
        ls = agg[:, _DLAT:]
        mu_ref[...] = mu
        ls_ref[...] = ls
        z_ref[...] = mu + eps_ref[...] * jnp.exp(ls)

    out = jax.ShapeDtypeStruct((_N, _DLAT), jnp.float32)
    return pl.pallas_call(
        body,
        grid=(_N // _BN,),
        in_specs=[
            pl.BlockSpec((_NC, _BN, _DHID), lambda i: (0, i, 0)),
            pl.BlockSpec((_BN, _DHID), lambda i: (i, 0)),
            pl.BlockSpec((_NC, _BN, _DEGW), lambda i: (0, i, 0)),
            pl.BlockSpec((_BN, _DLAT), lambda i: (i, 0)),
        ],
        out_specs=[
            pl.BlockSpec((_BN, _DLAT), lambda i: (i, 0)),
            pl.BlockSpec((_BN, _DLAT), lambda i: (i, 0)),
            pl.BlockSpec((_BN, _DLAT), lambda i: (i, 0)),
        ],
        out_shape=[out, out, out],
    )(s2a, hws2, dc, eps)


_BM = 200


def _decoder(z):
    def body(zr_ref, zc_ref, o_ref):
        o_ref[...] = lax.dot_general(
            zr_ref[...],
            zc_ref[...],
            (((1,), (1,)), ((), ())),
            preferred_element_type=jnp.float32,
        )

    return pl.pallas_call(
        body,
        grid=(_N // _BM,),
        in_specs=[
            pl.BlockSpec((_BM, _DLAT), lambda i: (i, 0)),
            pl.BlockSpec((_N, _DLAT), lambda i: (0, 0)),
        ],
        out_specs=pl.BlockSpec((_BM, _N), lambda i: (i, 0)),
        out_shape=jax.ShapeDtypeStruct((_N, _N), jnp.float32),
    )(z, z)


def kernel(x, edge_index, W_hidden, W_mu, W_logstd):
    src = edge_index[0]
    dst = edge_index[1]
    pad = _EPAD - _E
    dummy = _N + (jnp.arange(pad, dtype=jnp.int32) % (_NPAD - _N))
    srcf = jnp.concatenate([src, jnp.zeros((pad,), jnp.int32)])
    dstf = jnp.concatenate([dst, dummy])
    dstp = dstf.reshape(_NW, _NCHUNK, _CHUNK)
    srcp = srcf.reshape(_NW, _NCHUNK, _CHUNK)

    dc = _deg(dstp).reshape(_NC, _NPAD, _DEGW)
    hws1 = _enc1(x, W_hidden, dc)
    s1a = _prop128(hws1, srcp, dstp).reshape(_NC, _NPAD, _DHID)
    wcat = jnp.concatenate(
        [W_mu, W_logstd, jnp.zeros((_DHID, _DHID - 2 * _DLAT), jnp.float32)], axis=1
    )
    hws2 = _enc2(s1a, hws1, wcat, dc)
    s2a = _prop128(hws2, srcp, dstp).reshape(_NC, _NPAD, _DHID)
    eps = jax.random.normal(jax.random.key(1), (_N, _DLAT), jnp.float32)
    mu, logstd, z = _latent(s2a, hws2, dc, eps)
    adj = _decoder(z)
    return (adj, mu, logstd)


# speedup vs baseline: 2.1751x; 2.1751x over previous
import functools

import jax
import jax.numpy as jnp
from jax import lax
from jax.experimental import pallas as pl
from jax.experimental.pallas import tpu as pltpu
from jax.experimental.pallas import tpu_sc as plsc

_N = 10000
_E = 160000
_DIN = 128
_DHID = 128
_DLAT = 32

_NC = 2
_NS = 16
_NW = _NC * _NS
_CHUNK = 128
_NCHUNK = 40
_NCHALF = 20
_EPAD = _NW * _CHUNK * _NCHUNK
_EHALF = _EPAD // 2
_NPAD = 10240
_RPT = _NPAD // _NS
_ZR = 64
_DEGW = 16


_NBUF = 2


def _make_prop(d, nchunk):
    mesh = plsc.VectorSubcoreMesh(core_axis_name="c", subcore_axis_name="s")

    @functools.partial(
        pl.kernel,
        out_type=jax.ShapeDtypeStruct((_NC * _NPAD, d), jnp.float32),
        mesh=mesh,
        scratch_types=[
            pltpu.VMEM((nchunk, _CHUNK), jnp.int32),
            pltpu.VMEM((nchunk, _CHUNK), jnp.int32),
            pltpu.VMEM_SHARED((_NPAD, d), jnp.float32),
        ]
        + [pltpu.VMEM((_CHUNK, d), jnp.float32)] * _NBUF
        + [pltpu.SemaphoreType.DMA] * _NBUF,
    )
    def prop(hws_hbm, src_hbm, dst_hbm, out_hbm, src_v, dst_v, acc,
             *ring_and_sems):
        rows = ring_and_sems[:_NBUF]
        sems = ring_and_sems[_NBUF:]
        c = lax.axis_index("c")
        s = lax.axis_index("s")
        wid = c * _NS + s
        zero = jnp.zeros((16,), jnp.float32)
        for i in range(_CHUNK):
            for k in range(d // 16):
                rows[0][i, pl.ds(k * 16, 16)] = zero
        for t in range(_RPT // _CHUNK):
            pltpu.sync_copy(rows[0], acc.at[pl.ds(s * _RPT + t * _CHUNK, _CHUNK)])
        pltpu.sync_copy(src_hbm.at[wid], src_v)
        pltpu.sync_copy(dst_hbm.at[wid], dst_v)
        plsc.subcore_barrier()

        for b in range(_NBUF - 1):
            pltpu.async_copy(hws_hbm.at[src_v.at[b]], rows[b], sems[b])

        def group(g, carry):
            j0 = g * _NBUF
            for b in range(_NBUF):
                j = j0 + b
                jn = j + _NBUF - 1
                bn = (_NBUF - 1 + b) % _NBUF

                @pl.when(jn < nchunk)
                def _():
                    pltpu.async_copy(hws_hbm.at[src_v.at[jn]], rows[bn], sems[bn])

                pltpu.make_async_copy(
                    hws_hbm.at[src_v.at[j]], rows[b], sems[b]
                ).wait()
                pltpu.sync_copy(rows[b], acc.at[dst_v.at[j]], add=True)
            return carry

        lax.fori_loop(0, nchunk // _NBUF, group, 0)
        plsc.subcore_barrier()
        pltpu.sync_copy(
            acc.at[pl.ds(s * _RPT, _RPT)],
            out_hbm.at[pl.ds(c * _NPAD + s * _RPT, _RPT)],
        )

    return prop


_prop128 = _make_prop(_DHID, _NCHUNK)


def _make_deg():
    mesh = plsc.VectorSubcoreMesh(core_axis_name="c", subcore_axis_name="s")

    @functools.partial(
        pl.kernel,
        out_type=jax.ShapeDtypeStruct((_NC * _NPAD, _DEGW), jnp.float32),
        mesh=mesh,
        scratch_types=[
            pltpu.VMEM((_NCHUNK, _CHUNK), jnp.int32),
            pltpu.VMEM((_CHUNK, _DEGW), jnp.float32),
            pltpu.VMEM((_ZR, _DEGW), jnp.float32),
            pltpu.VMEM_SHARED((_NPAD, _DEGW), jnp.float32),
        ],
    )
    def deg(dst_hbm, out_hbm, dst_v, ones_v, zbuf, acc):
        c = lax.axis_index("c")
        s = lax.axis_index("s")
        wid = c * _NS + s
        one = jnp.full((16,), 1.0, jnp.float32)
        zero = jnp.zeros((16,), jnp.float32)
        for i in range(_CHUNK):
            ones_v[i, :] = one
        for i in range(_ZR):
            zbuf[i, :] = zero
        for t in range(_RPT // _ZR):
            pltpu.sync_copy(zbuf, acc.at[pl.ds(s * _RPT + t * _ZR, _ZR)])
        pltpu.sync_copy(dst_hbm.at[wid], dst_v)
        plsc.subcore_barrier()

        def body(j, carry):
            pltpu.sync_copy(ones_v, acc.at[dst_v.at[j]], add=True)
            return carry

        lax.fori_loop(0, _NCHUNK, body, 0)
        plsc.subcore_barrier()
        pltpu.sync_copy(
            acc.at[pl.ds(s * _RPT, _RPT)],
            out_hbm.at[pl.ds(c * _NPAD + s * _RPT, _RPT)],
        )

    return deg


_deg = _make_deg()

_BN = 1000


def _dinv_of(dc_ref):
    deg = dc_ref[0, :, 0:1] + dc_ref[1, :, 0:1] + 1.0
    return lax.rsqrt(deg)


def _enc1(x, w, dc):
    def body(x_ref, w_ref, dc_ref, o_ref):
        hw = jnp.dot(x_ref[...], w_ref[...], preferred_element_type=jnp.float32)
        o_ref[...] = hw * _dinv_of(dc_ref)

    return pl.pallas_call(
        body,
        grid=(_N // _BN,),
        in_specs=[
            pl.BlockSpec((_BN, _DIN), lambda i: (i, 0)),
            pl.BlockSpec((_DIN, _DHID), lambda i: (0, 0)),
            pl.BlockSpec((_NC, _BN, _DEGW), lambda i: (0, i, 0)),
        ],
        out_specs=pl.BlockSpec((_BN, _DHID), lambda i: (i, 0)),
        out_shape=jax.ShapeDtypeStruct((_N, _DHID), jnp.float32),
    )(x, w, dc)


def _enc2(s1a, hws1, wcat, dc):
    def body(sa_ref, hws1_ref, w_ref, dc_ref, o_ref):
        dinv = _dinv_of(dc_ref)
        h = jnp.maximum(dinv * (sa_ref[0] + sa_ref[1] + hws1_ref[...]), 0.0)
        o_ref[...] = dinv * jnp.dot(
            h, w_ref[...], preferred_element_type=jnp.float32
        )

    return pl.pallas_call(
        body,
        grid=(_N // _BN,),
        in_specs=[
            pl.BlockSpec((_NC, _BN, _DHID), lambda i: (0, i, 0)),
            pl.BlockSpec((_BN, _DHID), lambda i: (i, 0)),
            pl.BlockSpec((_DHID, _DHID), lambda i: (0, 0)),
            pl.BlockSpec((_NC, _BN, _DEGW), lambda i: (0, i, 0)),
        ],
        out_specs=pl.BlockSpec((_BN, _DHID), lambda i: (i, 0)),
        out_shape=jax.ShapeDtypeStruct((_N, _DHID), jnp.float32),
    )(s1a, hws1, wcat, dc)


def _latent(s2a, hws2, dc, eps):
    def body(sa_ref, hws2_ref, dc_ref, eps_ref, mu_ref, ls_ref, z_ref):
        dinv = _dinv_of(dc_ref)
        agg = dinv * (
            sa_ref[0, :, : 2 * _DLAT]
            + sa_ref[1, :, : 2 * _DLAT]
            + hws2_ref[:, : 2 * _DLAT]
        )
        mu = agg[:, :_DLAT]
        ls = agg[:, _DLAT:]
        mu_ref[...] = mu
        ls_ref[...] = ls
        z_ref[...] = mu + eps_ref[...] * jnp.exp(ls)

    out = jax.ShapeDtypeStruct((_N, _DLAT), jnp.float32)
    return pl.pallas_call(
        body,
        grid=(_N // _BN,),
        in_specs=[
            pl.BlockSpec((_NC, _BN, _DHID), lambda i: (0, i, 0)),
            pl.BlockSpec((_BN, _DHID), lambda i: (i, 0)),
            pl.BlockSpec((_NC, _BN, _DEGW), lambda i: (0, i, 0)),
            pl.BlockSpec((_BN, _DLAT), lambda i: (i, 0)),
        ],
        out_specs=[
            pl.BlockSpec((_BN, _DLAT), lambda i: (i, 0)),
            pl.BlockSpec((_BN, _DLAT), lambda i: (i, 0)),
            pl.BlockSpec((_BN, _DLAT), lambda i: (i, 0)),
        ],
        out_shape=[out, out, out],
    )(s2a, hws2, dc, eps)


_BM = 200


def _decoder(z):
    def body(zr_ref, zc_ref, o_ref):
        o_ref[...] = lax.dot_general(
            zr_ref[...],
            zc_ref[...],
            (((1,), (1,)), ((), ())),
            preferred_element_type=jnp.float32,
        )

    return pl.pallas_call(
        body,
        grid=(_N // _BM,),
        in_specs=[
            pl.BlockSpec((_BM, _DLAT), lambda i: (i, 0)),
            pl.BlockSpec((_N, _DLAT), lambda i: (0, 0)),
        ],
        out_specs=pl.BlockSpec((_BM, _N), lambda i: (i, 0)),
        out_shape=jax.ShapeDtypeStruct((_N, _N), jnp.float32),
    )(z, z)


def kernel(x, edge_index, W_hidden, W_mu, W_logstd):
    src = edge_index[0]
    dst = edge_index[1]
    pad = _EPAD - _E
    dummy = _N + (jnp.arange(pad, dtype=jnp.int32) % (_NPAD - _N))
    srcf = jnp.concatenate([src, src[:pad]])
    dstf = jnp.concatenate([dst, dummy])
    dstp = dstf.reshape(_NW, _NCHUNK, _CHUNK)
    srcp = srcf.reshape(_NW, _NCHUNK, _CHUNK)

    dc = _deg(dstp).reshape(_NC, _NPAD, _DEGW)
    hws1 = _enc1(x, W_hidden, dc)
    s1a = _prop128(hws1, srcp, dstp).reshape(_NC, _NPAD, _DHID)
    wcat = jnp.concatenate(
        [W_mu, W_logstd, jnp.zeros((_DHID, _DHID - 2 * _DLAT), jnp.float32)], axis=1
    )
    hws2 = _enc2(s1a, hws1, wcat, dc)
    s2a = _prop128(hws2, srcp, dstp).reshape(_NC, _NPAD, _DHID)
    eps = jax.random.normal(jax.random.key(1), (_N, _DLAT), jnp.float32)
    mu, logstd, z = _latent(s2a, hws2, dc, eps)
    adj = _decoder(z)
    return (adj, mu, logstd)


# speedup vs baseline: 2.1934x; 1.0084x over previous
import functools

import jax
import jax.numpy as jnp
from jax import lax
from jax.experimental import pallas as pl
from jax.experimental.pallas import tpu as pltpu
from jax.experimental.pallas import tpu_sc as plsc

_N = 10000
_E = 160000
_DIN = 128
_DHID = 128
_DLAT = 32

_NC = 2
_NS = 16
_NW = _NC * _NS
_CHUNK = 128
_NCHUNK = 40
_NCHALF = 20
_EPAD = _NW * _CHUNK * _NCHUNK
_EHALF = _EPAD // 2
_NPAD = 10240
_RPT = _NPAD // _NS
_ZR = 64
_DEGW = 16


_NBUF = 2


def _make_prop(d, sd, nchunk):
    mesh = plsc.VectorSubcoreMesh(core_axis_name="c", subcore_axis_name="s")
    crows_types = (
        [pltpu.VMEM((_CHUNK, sd), jnp.float32)] * _NBUF if sd < d else []
    )

    @functools.partial(
        pl.kernel,
        out_type=jax.ShapeDtypeStruct((_NC * _NPAD, sd), jnp.float32),
        mesh=mesh,
        scratch_types=[
            pltpu.VMEM((nchunk, _CHUNK), jnp.int32),
            pltpu.VMEM((nchunk, _CHUNK), jnp.int32),
            pltpu.VMEM_SHARED((_NPAD, sd), jnp.float32),
        ]
        + [pltpu.VMEM((_CHUNK, d), jnp.float32)] * _NBUF
        + crows_types
        + [pltpu.SemaphoreType.DMA] * _NBUF,
    )
    def prop(hws_hbm, src_hbm, dst_hbm, out_hbm, src_v, dst_v, acc,
             *ring_and_sems):
        rows = ring_and_sems[:_NBUF]
        if sd < d:
            crows = ring_and_sems[_NBUF:2 * _NBUF]
            sems = ring_and_sems[2 * _NBUF:]
        else:
            crows = rows
            sems = ring_and_sems[_NBUF:]
        c = lax.axis_index("c")
        s = lax.axis_index("s")
        wid = c * _NS + s
        zero = jnp.zeros((16,), jnp.float32)
        for i in range(_CHUNK):
            for k in range(sd // 16):
                crows[0][i, pl.ds(k * 16, 16)] = zero
        for t in range(_RPT // _CHUNK):
            pltpu.sync_copy(crows[0], acc.at[pl.ds(s * _RPT + t * _CHUNK, _CHUNK)])
        pltpu.sync_copy(src_hbm.at[wid], src_v)
        pltpu.sync_copy(dst_hbm.at[wid], dst_v)
        plsc.subcore_barrier()

        for b in range(_NBUF - 1):
            pltpu.async_copy(hws_hbm.at[src_v.at[b]], rows[b], sems[b])

        def group(g, carry):
            j0 = g * _NBUF
            for b in range(_NBUF):
                j = j0 + b
                jn = j + _NBUF - 1
                bn = (_NBUF - 1 + b) % _NBUF

                @pl.when(jn < nchunk)
                def _():
                    pltpu.async_copy(hws_hbm.at[src_v.at[jn]], rows[bn], sems[bn])

                pltpu.make_async_copy(
                    hws_hbm.at[src_v.at[j]], rows[b], sems[b]
                ).wait()
                if sd < d:
                    for r in range(_CHUNK):
                        for k in range(sd // 16):
                            crows[b][r, pl.ds(k * 16, 16)] = rows[b][
                                r, pl.ds(k * 16, 16)
                            ]
                pltpu.sync_copy(crows[b], acc.at[dst_v.at[j]], add=True)
            return carry

        lax.fori_loop(0, nchunk // _NBUF, group, 0)
        plsc.subcore_barrier()
        pltpu.sync_copy(
            acc.at[pl.ds(s * _RPT, _RPT)],
            out_hbm.at[pl.ds(c * _NPAD + s * _RPT, _RPT)],
        )

    return prop


_prop128 = _make_prop(_DHID, _DHID, _NCHUNK)
_prop64 = _make_prop(_DHID, 2 * _DLAT, _NCHUNK)


def _make_deg():
    mesh = plsc.VectorSubcoreMesh(core_axis_name="c", subcore_axis_name="s")

    @functools.partial(
        pl.kernel,
        out_type=jax.ShapeDtypeStruct((_NC * _NPAD, _DEGW), jnp.float32),
        mesh=mesh,
        scratch_types=[
            pltpu.VMEM((_NCHUNK, _CHUNK), jnp.int32),
            pltpu.VMEM((_CHUNK, _DEGW), jnp.float32),
            pltpu.VMEM((_ZR, _DEGW), jnp.float32),
            pltpu.VMEM_SHARED((_NPAD, _DEGW), jnp.float32),
        ],
    )
    def deg(dst_hbm, out_hbm, dst_v, ones_v, zbuf, acc):
        c = lax.axis_index("c")
        s = lax.axis_index("s")
        wid = c * _NS + s
        one = jnp.full((16,), 1.0, jnp.float32)
        zero = jnp.zeros((16,), jnp.float32)
        for i in range(_CHUNK):
            ones_v[i, :] = one
        for i in range(_ZR):
            zbuf[i, :] = zero
        for t in range(_RPT // _ZR):
            pltpu.sync_copy(zbuf, acc.at[pl.ds(s * _RPT + t * _ZR, _ZR)])
        pltpu.sync_copy(dst_hbm.at[wid], dst_v)
        plsc.subcore_barrier()

        def body(j, carry):
            pltpu.sync_copy(ones_v, acc.at[dst_v.at[j]], add=True)
            return carry

        lax.fori_loop(0, _NCHUNK, body, 0)
        plsc.subcore_barrier()
        pltpu.sync_copy(
            acc.at[pl.ds(s * _RPT, _RPT)],
            out_hbm.at[pl.ds(c * _NPAD + s * _RPT, _RPT)],
        )

    return deg


_deg = _make_deg()

_BN = 1000


def _dinv_of(dc_ref):
    deg = dc_ref[0, :, 0:1] + dc_ref[1, :, 0:1] + 1.0
    return lax.rsqrt(deg)


def _enc1(x, w, dc):
    def body(x_ref, w_ref, dc_ref, o_ref):
        hw = jnp.dot(x_ref[...], w_ref[...], preferred_element_type=jnp.float32)
        o_ref[...] = hw * _dinv_of(dc_ref)

    return pl.pallas_call(
        body,
        grid=(_N // _BN,),
        in_specs=[
            pl.BlockSpec((_BN, _DIN), lambda i: (i, 0)),
            pl.BlockSpec((_DIN, _DHID), lambda i: (0, 0)),
            pl.BlockSpec((_NC, _BN, _DEGW), lambda i: (0, i, 0)),
        ],
        out_specs=pl.BlockSpec((_BN, _DHID), lambda i: (i, 0)),
        out_shape=jax.ShapeDtypeStruct((_N, _DHID), jnp.float32),
    )(x, w, dc)


def _enc2(s1a, hws1, wcat, dc):
    def body(sa_ref, hws1_ref, w_ref, dc_ref, o_ref):
        dinv = _dinv_of(dc_ref)
        h = jnp.maximum(dinv * (sa_ref[0] + sa_ref[1] + hws1_ref[...]), 0.0)
        o_ref[...] = dinv * jnp.dot(
            h, w_ref[...], preferred_element_type=jnp.float32
        )

    return pl.pallas_call(
        body,
        grid=(_N // _BN,),
        in_specs=[
            pl.BlockSpec((_NC, _BN, _DHID), lambda i: (0, i, 0)),
            pl.BlockSpec((_BN, _DHID), lambda i: (i, 0)),
            pl.BlockSpec((_DHID, _DHID), lambda i: (0, 0)),
            pl.BlockSpec((_NC, _BN, _DEGW), lambda i: (0, i, 0)),
        ],
        out_specs=pl.BlockSpec((_BN, _DHID), lambda i: (i, 0)),
        out_shape=jax.ShapeDtypeStruct((_N, _DHID), jnp.float32),
    )(s1a, hws1, wcat, dc)


def _latent(s2a, hws2, dc, eps):
    def body(sa_ref, hws2_ref, dc_ref, eps_ref, mu_ref, ls_ref, z_ref):
        dinv = _dinv_of(dc_ref)
        agg = dinv * (
            sa_ref[0] + sa_ref[1] + hws2_ref[:, : 2 * _DLAT]
        )
        mu = agg[:, :_DLAT]
        ls = agg[:, _DLAT:]
        mu_ref[...] = mu
        ls_ref[...] = ls
        z_ref[...] = mu + eps_ref[...] * jnp.exp(ls)

    out = jax.ShapeDtypeStruct((_N, _DLAT), jnp.float32)
    return pl.pallas_call(
        body,
        grid=(_N // _BN,),
        in_specs=[
            pl.BlockSpec((_NC, _BN, 2 * _DLAT), lambda i: (0, i, 0)),
            pl.BlockSpec((_BN, _DHID), lambda i: (i, 0)),
            pl.BlockSpec((_NC, _BN, _DEGW), lambda i: (0, i, 0)),
            pl.BlockSpec((_BN, _DLAT), lambda i: (i, 0)),
        ],
        out_specs=[
            pl.BlockSpec((_BN, _DLAT), lambda i: (i, 0)),
            pl.BlockSpec((_BN, _DLAT), lambda i: (i, 0)),
            pl.BlockSpec((_BN, _DLAT), lambda i: (i, 0)),
        ],
        out_shape=[out, out, out],
    )(s2a, hws2, dc, eps)


_BM = 200


def _decoder(z):
    def body(zr_ref, zc_ref, o_ref):
        o_ref[...] = lax.dot_general(
            zr_ref[...],
            zc_ref[...],
            (((1,), (1,)), ((), ())),
            preferred_element_type=jnp.float32,
        )

    return pl.pallas_call(
        body,
        grid=(_N // _BM,),
        in_specs=[
            pl.BlockSpec((_BM, _DLAT), lambda i: (i, 0)),
            pl.BlockSpec((_N, _DLAT), lambda i: (0, 0)),
        ],
        out_specs=pl.BlockSpec((_BM, _N), lambda i: (i, 0)),
        out_shape=jax.ShapeDtypeStruct((_N, _N), jnp.float32),
    )(z, z)


def kernel(x, edge_index, W_hidden, W_mu, W_logstd):
    src = edge_index[0]
    dst = edge_index[1]
    pad = _EPAD - _E
    dummy = _N + (jnp.arange(pad, dtype=jnp.int32) % (_NPAD - _N))
    srcf = jnp.concatenate([src, src[:pad]])
    dstf = jnp.concatenate([dst, dummy])
    dstp = dstf.reshape(_NW, _NCHUNK, _CHUNK)
    srcp = srcf.reshape(_NW, _NCHUNK, _CHUNK)

    dc = _deg(dstp).reshape(_NC, _NPAD, _DEGW)
    hws1 = _enc1(x, W_hidden, dc)
    s1a = _prop128(hws1, srcp, dstp).reshape(_NC, _NPAD, _DHID)
    wcat = jnp.concatenate(
        [W_mu, W_logstd, jnp.zeros((_DHID, _DHID - 2 * _DLAT), jnp.float32)], axis=1
    )
    hws2 = _enc2(s1a, hws1, wcat, dc)
    s2a = _prop64(hws2, srcp, dstp).reshape(_NC, _NPAD, 2 * _DLAT)
    eps = jax.random.normal(jax.random.key(1), (_N, _DLAT), jnp.float32)
    mu, logstd, z = _latent(s2a, hws2, dc, eps)
    adj = _decoder(z)
    return (adj, mu, logstd)
